# baseline (speedup 1.0000x reference)
import jax
import jax.numpy as jnp
from jax.experimental import pallas as pl
from jax.experimental.pallas import tpu as pltpu

N = 10000
M = 2048
BN = 1000
EPS = 1e-9


def _pass1_kernel(h_ref, x_ref, z_ref, w_ref,
                  psi_W_ref, psi_b_ref, phi_W_ref, phi_b_ref,
                  g1_W_ref, g1_b_ref, g2_W_ref, g2_b_ref,
                  c1_W_ref, c1_b_ref,
                  g_ref, s_ref, de_ref, m1_ref, h16_ref):
    i = pl.program_id(0)

    @pl.when(i == 0)
    def _init():
        de_ref[...] = jnp.zeros_like(de_ref)
        m1_ref[...] = jnp.zeros_like(m1_ref)

    h16 = h_ref[...].astype(jnp.bfloat16)
    h16_ref[...] = h16

    dv = jnp.dot(h16, w_ref[...],
                 preferred_element_type=jnp.float32)
    s = jax.lax.rsqrt(dv + EPS)
    s_ref[...] = s
    ones = jnp.ones((1, BN), jnp.bfloat16)
    de_ref[...] += jnp.dot(ones, h16,
                           preferred_element_type=jnp.float32)

    x1 = x_ref[...] @ psi_W_ref[...] + psi_b_ref[...]
    z1 = z_ref[...] @ phi_W_ref[...] + phi_b_ref[...]
    cat = jnp.concatenate([x1, z1], axis=1)
    gh = jnp.maximum(cat @ g1_W_ref[...] + g1_b_ref[...], 0.0)
    g = jax.nn.sigmoid(gh @ g2_W_ref[...] + g2_b_ref[...])
    g_ref[...] = g
    fused = g * z1 + (1.0 - g) * x1

    x1c = fused @ c1_W_ref[...] + c1_b_ref[...]
    xn1 = (x1c * s).astype(jnp.bfloat16)
    m1_ref[...] += jax.lax.dot_general(
        h16, xn1, (((0,), (0,)), ((), ())),
        preferred_element_type=jnp.float32)


def _pass2_kernel(h_ref, s_ref, m1_ref, w_ref, de_ref,
                  c2_W_ref, c2_b_ref, m2_ref):
    i = pl.program_id(0)

    @pl.when(i == 0)
    def _init():
        m2_ref[...] = jnp.zeros_like(m2_ref)

    h16 = h_ref[...]
    s = s_ref[...]
    m1n = (m1_ref[...] * (w_ref[...] / (de_ref[...] + EPS))
           ).astype(jnp.bfloat16)
    y1 = jnp.dot(h16, m1n, preferred_element_type=jnp.float32)
    h1 = jnp.maximum(y1 * s, 0.0)
    x2 = h1 @ c2_W_ref[...] + c2_b_ref[...]
    xn2 = (x2 * s).astype(jnp.bfloat16)
    m2_ref[...] += jax.lax.dot_general(
        h16, xn2, (((0,), (0,)), ((), ())),
        preferred_element_type=jnp.float32)


def _pass3_kernel(h_ref, s_ref, m2_ref, w_ref, de_ref,
                  hd_W_ref, hd_b_ref, out_ref):
    h16 = h_ref[...]
    s = s_ref[...]
    m2n = (m2_ref[...] * (w_ref[...] / (de_ref[...] + EPS))
           ).astype(jnp.bfloat16)
    y2 = jnp.dot(h16, m2n, preferred_element_type=jnp.float32)
    h2 = jnp.maximum(y2 * s, 0.0)
    out_ref[...] = h2 @ hd_W_ref[...] + hd_b_ref[...]


def _full(shape):
    nd = len(shape)
    return pl.BlockSpec(shape, lambda i: (0,) * nd)


def kernel(x, z, H, w,
           psi_W, psi_b, phi_W, phi_b,
           g1_W, g1_b, g2_W, g2_b,
           c1_W, c1_b, c2_W, c2_b,
           hd_W, hd_b):
    grid = (N // BN,)
    w_col16 = w.reshape(M, 1).astype(jnp.bfloat16)
    w_col = w.reshape(M, 1)

    params = dict(
        grid=grid,
        compiler_params=pltpu.CompilerParams(
            dimension_semantics=("arbitrary",)),
    )

    row = lambda shape: pl.BlockSpec(shape, lambda i: (i, 0))

    g, s, de, m1, h16 = pl.pallas_call(
        _pass1_kernel,
        in_specs=[row((BN, M)), row((BN, x.shape[1])), row((BN, z.shape[1])),
                  _full((M, 1)),
                  _full(psi_W.shape), _full((1, psi_b.shape[0])),
                  _full(phi_W.shape), _full((1, phi_b.shape[0])),
                  _full(g1_W.shape), _full((1, g1_b.shape[0])),
                  _full(g2_W.shape), _full((1, g2_b.shape[0])),
                  _full(c1_W.shape), _full((1, c1_b.shape[0]))],
        out_specs=[row((BN, 32)), row((BN, 1)), _full((1, M)), _full((M, 64)),
                   row((BN, M))],
        out_shape=[jax.ShapeDtypeStruct((N, 32), jnp.float32),
                   jax.ShapeDtypeStruct((N, 1), jnp.float32),
                   jax.ShapeDtypeStruct((1, M), jnp.float32),
                   jax.ShapeDtypeStruct((M, 64), jnp.float32),
                   jax.ShapeDtypeStruct((N, M), jnp.bfloat16)],
        **params,
    )(H, x, z, w_col16,
      psi_W, psi_b.reshape(1, -1), phi_W, phi_b.reshape(1, -1),
      g1_W, g1_b.reshape(1, -1), g2_W, g2_b.reshape(1, -1),
      c1_W, c1_b.reshape(1, -1))

    de_col = de.reshape(M, 1)

    m2 = pl.pallas_call(
        _pass2_kernel,
        in_specs=[row((BN, M)), row((BN, 1)), _full((M, 64)),
                  _full((M, 1)), _full((M, 1)),
                  _full(c2_W.shape), _full((1, c2_b.shape[0]))],
        out_specs=_full((M, 64)),
        out_shape=jax.ShapeDtypeStruct((M, 64), jnp.float32),
        **params,
    )(h16, s, m1, w_col, de_col, c2_W, c2_b.reshape(1, -1))

    logits = pl.pallas_call(
        _pass3_kernel,
        in_specs=[row((BN, M)), row((BN, 1)), _full((M, 64)),
                  _full((M, 1)), _full((M, 1)),
                  _full(hd_W.shape), _full((1, hd_b.shape[0]))],
        out_specs=row((BN, hd_b.shape[0])),
        out_shape=jax.ShapeDtypeStruct((N, hd_b.shape[0]), jnp.float32),
        **params,
    )(h16, s, m2, w_col, de_col, hd_W, hd_b.reshape(1, -1))

    return (logits, g)


# speedup vs baseline: 1.2248x; 1.2248x over previous
import jax
import jax.numpy as jnp
from jax.experimental import pallas as pl
from jax.experimental.pallas import tpu as pltpu

N = 10000
M = 2048
BN1 = 1000
BN2 = 2000
EPS = 1e-9


def _pass1_kernel(h_ref, x_ref, z_ref, w_ref,
                  psi_W_ref, psi_b_ref, phi_W_ref, phi_b_ref,
                  g1_W_ref, g1_b_ref, g2_W_ref, g2_b_ref,
                  c1_W_ref, c1_b_ref,
                  g_ref, s_ref, de_ref, m1t_ref, h16_ref):
    i = pl.program_id(0)

    @pl.when(i == 0)
    def _init():
        de_ref[...] = jnp.zeros_like(de_ref)
        m1t_ref[...] = jnp.zeros_like(m1t_ref)

    h16 = h_ref[...].astype(jnp.bfloat16)
    h16_ref[...] = h16

    dv = jnp.dot(h16, w_ref[...],
                 preferred_element_type=jnp.float32)
    s = jax.lax.rsqrt(dv + EPS)
    s_ref[...] = s
    ones = jnp.ones((1, BN1), jnp.bfloat16)
    de_ref[...] += jnp.dot(ones, h16,
                           preferred_element_type=jnp.float32)

    x1 = x_ref[...] @ psi_W_ref[...] + psi_b_ref[...]
    z1 = z_ref[...] @ phi_W_ref[...] + phi_b_ref[...]
    cat = jnp.concatenate([x1, z1], axis=1)
    gh = jnp.maximum(cat @ g1_W_ref[...] + g1_b_ref[...], 0.0)
    g = jax.nn.sigmoid(gh @ g2_W_ref[...] + g2_b_ref[...])
    g_ref[...] = g
    fused = g * z1 + (1.0 - g) * x1

    x1c = fused @ c1_W_ref[...] + c1_b_ref[...]
    xn1 = (x1c * s).astype(jnp.bfloat16)
    m1t_ref[...] += jax.lax.dot_general(
        xn1, h16, (((0,), (0,)), ((), ())),
        preferred_element_type=jnp.float32)


def _pass2_kernel(h_ref, s_ref, m1n_ref, c2_W_ref, c2_b_ref, m2t_ref):
    i = pl.program_id(0)

    @pl.when(i == 0)
    def _init():
        m2t_ref[...] = jnp.zeros_like(m2t_ref)

    h16 = h_ref[...]
    s = s_ref[...]
    y1 = jnp.dot(h16, m1n_ref[...],
                 preferred_element_type=jnp.float32)
    h1 = jnp.maximum(y1 * s, 0.0)
    x2 = h1 @ c2_W_ref[...] + c2_b_ref[...]
    xn2 = (x2 * s).astype(jnp.bfloat16)
    m2t_ref[...] += jax.lax.dot_general(
        xn2, h16, (((0,), (0,)), ((), ())),
        preferred_element_type=jnp.float32)


def _pass3_kernel(h_ref, s_ref, m2n_ref, hd_W_ref, hd_b_ref, out_ref):
    h16 = h_ref[...]
    s = s_ref[...]
    y2 = jnp.dot(h16, m2n_ref[...],
                 preferred_element_type=jnp.float32)
    h2 = jnp.maximum(y2 * s, 0.0)
    out_ref[...] = h2 @ hd_W_ref[...] + hd_b_ref[...]


def _full(shape):
    nd = len(shape)
    return pl.BlockSpec(shape, lambda i: (0,) * nd)


def kernel(x, z, H, w,
           psi_W, psi_b, phi_W, phi_b,
           g1_W, g1_b, g2_W, g2_b,
           c1_W, c1_b, c2_W, c2_b,
           hd_W, hd_b):
    w_col16 = w.reshape(M, 1).astype(jnp.bfloat16)

    seq = dict(compiler_params=pltpu.CompilerParams(
        dimension_semantics=("arbitrary",)))

    row = lambda shape: pl.BlockSpec(shape, lambda i: (i, 0))

    g, s, de, m1t, h16 = pl.pallas_call(
        _pass1_kernel,
        grid=(N // BN1,),
        in_specs=[row((BN1, M)), row((BN1, x.shape[1])), row((BN1, z.shape[1])),
                  _full((M, 1)),
                  _full(psi_W.shape), _full((1, psi_b.shape[0])),
                  _full(phi_W.shape), _full((1, phi_b.shape[0])),
                  _full(g1_W.shape), _full((1, g1_b.shape[0])),
                  _full(g2_W.shape), _full((1, g2_b.shape[0])),
                  _full(c1_W.shape), _full((1, c1_b.shape[0]))],
        out_specs=[row((BN1, 32)), row((BN1, 1)), _full((1, M)),
                   _full((64, M)), row((BN1, M))],
        out_shape=[jax.ShapeDtypeStruct((N, 32), jnp.float32),
                   jax.ShapeDtypeStruct((N, 1), jnp.float32),
                   jax.ShapeDtypeStruct((1, M), jnp.float32),
                   jax.ShapeDtypeStruct((64, M), jnp.float32),
                   jax.ShapeDtypeStruct((N, M), jnp.bfloat16)],
        **seq,
    )(H, x, z, w_col16,
      psi_W, psi_b.reshape(1, -1), phi_W, phi_b.reshape(1, -1),
      g1_W, g1_b.reshape(1, -1), g2_W, g2_b.reshape(1, -1),
      c1_W, c1_b.reshape(1, -1))

    se = (w / (de.reshape(M) + EPS))[None, :]
    m1n = (m1t * se).T.astype(jnp.bfloat16)

    m2t = pl.pallas_call(
        _pass2_kernel,
        grid=(N // BN2,),
        in_specs=[row((BN2, M)), row((BN2, 1)), _full((M, 64)),
                  _full(c2_W.shape), _full((1, c2_b.shape[0]))],
        out_specs=_full((64, M)),
        out_shape=jax.ShapeDtypeStruct((64, M), jnp.float32),
        **seq,
    )(h16, s, m1n, c2_W, c2_b.reshape(1, -1))

    m2n = (m2t * se).T.astype(jnp.bfloat16)

    logits = pl.pallas_call(
        _pass3_kernel,
        grid=(N // BN2,),
        in_specs=[row((BN2, M)), row((BN2, 1)), _full((M, 64)),
                  _full(hd_W.shape), _full((1, hd_b.shape[0]))],
        out_specs=row((BN2, hd_b.shape[0])),
        out_shape=jax.ShapeDtypeStruct((N, hd_b.shape[0]), jnp.float32),
        **seq,
    )(h16, s, m2n, hd_W, hd_b.reshape(1, -1))

    return (logits, g)


# speedup vs baseline: 1.2475x; 1.0186x over previous
import jax
import jax.numpy as jnp
from jax.experimental import pallas as pl
from jax.experimental.pallas import tpu as pltpu

N = 10000
M = 2048
BN1 = 1000
BN2 = 2000
EPS = 1e-9


def _pass1_kernel(h_ref, x_ref, z_ref, w_ref,
                  psi_W_ref, psi_b_ref, phi_W_ref, phi_b_ref,
                  g1_W_ref, g1_b_ref, g2_W_ref, g2_b_ref,
                  c1_W_ref, c1_b_ref,
                  g_ref, s_ref, de_ref, m1t_ref, h16_ref):
    i = pl.program_id(0)

    @pl.when(i == 0)
    def _init():
        de_ref[...] = jnp.zeros_like(de_ref)
        m1t_ref[...] = jnp.zeros_like(m1t_ref)

    h16 = h_ref[...].astype(jnp.bfloat16)
    h16_ref[...] = h16

    dv = jnp.dot(h16, w_ref[...],
                 preferred_element_type=jnp.float32)
    s = jax.lax.rsqrt(dv + EPS)
    s_ref[...] = s
    ones = jnp.ones((1, BN1), jnp.bfloat16)
    de_ref[...] += jnp.dot(ones, h16,
                           preferred_element_type=jnp.float32)

    x1 = x_ref[...] @ psi_W_ref[...] + psi_b_ref[...]
    z1 = z_ref[...] @ phi_W_ref[...] + phi_b_ref[...]
    cat = jnp.concatenate([x1, z1], axis=1)
    gh = jnp.maximum(cat @ g1_W_ref[...] + g1_b_ref[...], 0.0)
    g = jax.nn.sigmoid(gh @ g2_W_ref[...] + g2_b_ref[...])
    g_ref[...] = g
    fused = g * z1 + (1.0 - g) * x1

    x1c = fused @ c1_W_ref[...] + c1_b_ref[...]
    xn1 = (x1c * s).astype(jnp.bfloat16)
    m1t_ref[...] += jax.lax.dot_general(
        xn1, h16, (((0,), (0,)), ((), ())),
        preferred_element_type=jnp.float32)


def _pass2_kernel(h_ref, s_ref, m1t_ref, w_ref, de_ref,
                  c2_W_ref, c2_b_ref, m2t_ref, m1n_scr):
    i = pl.program_id(0)

    @pl.when(i == 0)
    def _init():
        m2t_ref[...] = jnp.zeros_like(m2t_ref)
        se = w_ref[...] / (de_ref[...] + EPS)
        m1n_scr[...] = jnp.transpose(
            (m1t_ref[...] * se).astype(jnp.bfloat16))

    h16 = h_ref[...]
    s = s_ref[...]
    y1 = jnp.dot(h16, m1n_scr[...],
                 preferred_element_type=jnp.float32)
    h1 = jnp.maximum(y1 * s, 0.0)
    x2 = h1 @ c2_W_ref[...] + c2_b_ref[...]
    xn2 = (x2 * s).astype(jnp.bfloat16)
    m2t_ref[...] += jax.lax.dot_general(
        xn2, h16, (((0,), (0,)), ((), ())),
        preferred_element_type=jnp.float32)


def _pass3_kernel(h_ref, s_ref, m2t_ref, w_ref, de_ref,
                  hd_W_ref, hd_b_ref, out_ref, m2n_scr):
    i = pl.program_id(0)

    @pl.when(i == 0)
    def _init():
        se = w_ref[...] / (de_ref[...] + EPS)
        m2n_scr[...] = jnp.transpose(
            (m2t_ref[...] * se).astype(jnp.bfloat16))

    h16 = h_ref[...]
    s = s_ref[...]
    y2 = jnp.dot(h16, m2n_scr[...],
                 preferred_element_type=jnp.float32)
    h2 = jnp.maximum(y2 * s, 0.0)
    out_ref[...] = h2 @ hd_W_ref[...] + hd_b_ref[...]


def _full(shape):
    nd = len(shape)
    return pl.BlockSpec(shape, lambda i: (0,) * nd)


def kernel(x, z, H, w,
           psi_W, psi_b, phi_W, phi_b,
           g1_W, g1_b, g2_W, g2_b,
           c1_W, c1_b, c2_W, c2_b,
           hd_W, hd_b):
    w_col16 = w.reshape(M, 1).astype(jnp.bfloat16)

    seq = dict(compiler_params=pltpu.CompilerParams(
        dimension_semantics=("arbitrary",)))

    row = lambda shape: pl.BlockSpec(shape, lambda i: (i, 0))

    g, s, de, m1t, h16 = pl.pallas_call(
        _pass1_kernel,
        grid=(N // BN1,),
        in_specs=[row((BN1, M)), row((BN1, x.shape[1])), row((BN1, z.shape[1])),
                  _full((M, 1)),
                  _full(psi_W.shape), _full((1, psi_b.shape[0])),
                  _full(phi_W.shape), _full((1, phi_b.shape[0])),
                  _full(g1_W.shape), _full((1, g1_b.shape[0])),
                  _full(g2_W.shape), _full((1, g2_b.shape[0])),
                  _full(c1_W.shape), _full((1, c1_b.shape[0]))],
        out_specs=[row((BN1, 32)), row((BN1, 1)), _full((1, M)),
                   _full((64, M)), row((BN1, M))],
        out_shape=[jax.ShapeDtypeStruct((N, 32), jnp.float32),
                   jax.ShapeDtypeStruct((N, 1), jnp.float32),
                   jax.ShapeDtypeStruct((1, M), jnp.float32),
                   jax.ShapeDtypeStruct((64, M), jnp.float32),
                   jax.ShapeDtypeStruct((N, M), jnp.bfloat16)],
        **seq,
    )(H, x, z, w_col16,
      psi_W, psi_b.reshape(1, -1), phi_W, phi_b.reshape(1, -1),
      g1_W, g1_b.reshape(1, -1), g2_W, g2_b.reshape(1, -1),
      c1_W, c1_b.reshape(1, -1))

    w_row = w.reshape(1, M)

    m2t = pl.pallas_call(
        _pass2_kernel,
        grid=(N // BN2,),
        in_specs=[row((BN2, M)), row((BN2, 1)), _full((64, M)),
                  _full((1, M)), _full((1, M)),
                  _full(c2_W.shape), _full((1, c2_b.shape[0]))],
        out_specs=_full((64, M)),
        out_shape=jax.ShapeDtypeStruct((64, M), jnp.float32),
        scratch_shapes=[pltpu.VMEM((M, 64), jnp.bfloat16)],
        **seq,
    )(h16, s, m1t, w_row, de, c2_W, c2_b.reshape(1, -1))

    logits = pl.pallas_call(
        _pass3_kernel,
        grid=(N // BN2,),
        in_specs=[row((BN2, M)), row((BN2, 1)), _full((64, M)),
                  _full((1, M)), _full((1, M)),
                  _full(hd_W.shape), _full((1, hd_b.shape[0]))],
        out_specs=row((BN2, hd_b.shape[0])),
        out_shape=jax.ShapeDtypeStruct((N, hd_b.shape[0]), jnp.float32),
        scratch_shapes=[pltpu.VMEM((M, 64), jnp.bfloat16)],
        **seq,
    )(h16, s, m2t, w_row, de, hd_W, hd_b.reshape(1, -1))

    return (logits, g)
